# speedup vs baseline: 4.2873x; 4.2873x over previous
import functools

import jax
import jax.numpy as jnp
from jax import lax
from jax.experimental import pallas as pl
from jax.experimental.pallas import tpu as pltpu
from jax.experimental.pallas import tpu_sc as plsc

NC = 2
NS = 16
CHUNK = 128


def _matmul(X, W):
    n, d_in = X.shape
    d_out = W.shape[1]
    bm = 2000 if n % 2000 == 0 else n
    grid = (n // bm,)

    def mm_body(x_ref, w_ref, o_ref):
        o_ref[...] = jnp.dot(x_ref[...], w_ref[...],
                             preferred_element_type=jnp.float32)

    return pl.pallas_call(
        mm_body,
        grid=grid,
        in_specs=[
            pl.BlockSpec((bm, d_in), lambda i: (i, 0)),
            pl.BlockSpec((d_in, d_out), lambda i: (0, 0)),
        ],
        out_specs=pl.BlockSpec((bm, d_out), lambda i: (i, 0)),
        out_shape=jax.ShapeDtypeStruct((n, d_out), jnp.float32),
    )(X, W)


def _add(a, b):
    n, d = a.shape
    bm = 2048 if n % 2048 == 0 else n
    grid = (n // bm,)

    def add_body(a_ref, b_ref, o_ref):
        o_ref[...] = a_ref[...] + b_ref[...]

    return pl.pallas_call(
        add_body,
        grid=grid,
        in_specs=[
            pl.BlockSpec((bm, d), lambda i: (i, 0)),
            pl.BlockSpec((bm, d), lambda i: (i, 0)),
        ],
        out_specs=pl.BlockSpec((bm, d), lambda i: (i, 0)),
        out_shape=jax.ShapeDtypeStruct((n, d), jnp.float32),
    )(a, b)


def _make_sc_scatter(n_acc, d, cpw):
    mesh = plsc.VectorSubcoreMesh(core_axis_name="c", subcore_axis_name="s",
                                  num_cores=NC, num_subcores=NS)
    zpt = n_acc // (NS * CHUNK)

    @functools.partial(
        pl.kernel,
        out_type=jax.ShapeDtypeStruct((NC * n_acc, d), jnp.float32),
        mesh=mesh,
        scratch_types=[
            pltpu.VMEM((CHUNK,), jnp.int32),
            pltpu.VMEM((CHUNK,), jnp.int32),
            pltpu.VMEM((CHUNK, d), jnp.float32),
            pltpu.VMEM_SHARED((n_acc, d), jnp.float32),
            pltpu.SemaphoreType.DMA,
        ],
    )
    def sc_kernel(comb_hbm, col_hbm, row_hbm, out_hbm,
                  col_v, row_v, rows_v, acc_sh, sem):
        c = lax.axis_index("c")
        s = lax.axis_index("s")
        wid = c * NS + s

        def zrow(i, carry):
            for j in range(d // 16):
                rows_v[i, pl.ds(j * 16, 16)] = jnp.zeros((16,), jnp.float32)
            return carry
        lax.fori_loop(0, CHUNK, zrow, 0)

        def zcopy(j, carry):
            base = (s * zpt + j) * CHUNK
            pltpu.sync_copy(rows_v, acc_sh.at[pl.ds(base, CHUNK)])
            return carry
        lax.fori_loop(0, zpt, zcopy, 0)
        plsc.subcore_barrier()

        def echunk(k, carry):
            base = (wid * cpw + k) * CHUNK
            pltpu.sync_copy(col_hbm.at[pl.ds(base, CHUNK)], col_v)
            pltpu.async_copy(comb_hbm.at[col_v], rows_v, sem).wait()
            pltpu.sync_copy(row_hbm.at[pl.ds(base, CHUNK)], row_v)
            pltpu.sync_copy(rows_v, acc_sh.at[row_v], add=True)
            return carry
        lax.fori_loop(0, cpw, echunk, 0)
        plsc.subcore_barrier()

        def wout(j, carry):
            base = (s * zpt + j) * CHUNK
            pltpu.sync_copy(acc_sh.at[pl.ds(base, CHUNK)],
                            out_hbm.at[pl.ds(c * n_acc + base, CHUNK)])
            return carry
        lax.fori_loop(0, zpt, wout, 0)

    return sc_kernel


@jax.jit
def kernel(X, edge_index, W):
    n, _ = X.shape
    d = W.shape[1]
    e = edge_index.shape[1]

    nw = NC * NS
    cpw = -(-e // (CHUNK * nw))
    e_pad = cpw * nw * CHUNK
    n_acc = -(-(n + 1) // (NS * CHUNK)) * (NS * CHUNK)

    row = edge_index[0]
    col = edge_index[1]
    pad = e_pad - e
    if pad:
        col = jnp.concatenate([col, jnp.zeros((pad,), jnp.int32)])
        row = jnp.concatenate([row, jnp.full((pad,), n, jnp.int32)])

    comb = _matmul(X, W)
    partials = _make_sc_scatter(n_acc, d, cpw)(comb, col, row)
    out = _add(partials[:n_acc], partials[n_acc:2 * n_acc])
    return out[:n]
